# baseline (speedup 1.0000x reference)
import functools
import jax
import jax.numpy as jnp
from jax import lax
from jax.experimental import pallas as pl
from jax.experimental.pallas import tpu as pltpu
from jax.experimental.pallas import tpu_sc as plsc

NUM_CHARS = 256
EMB_DIM = 16
BATCH = 16384
MSG_LEN = 256
TOTAL = BATCH * MSG_LEN

NC = 2
NS = 16
NW = NC * NS
L = 16

G = 128
CHUNK = 2048
NG = CHUNK // G
NVEC = CHUNK // L
UNROLL = 4
IDX_PER_W = TOTAL // NW
STEPS = IDX_PER_W // CHUNK
NPAIR = STEPS // 2


def _sc_body(x1_hbm, tab_hbm, tabT_hbm, out_hbm,
             tabT_v, tabS,
             idxt_a, idxt_b, idxs_a, idxs_b, rows_t, rows_s,
             sem_i, sem_is, sem_g, sem_ot, sem_os):
    sid = lax.axis_index("s")
    wid = sid * NC + lax.axis_index("c")
    base0 = wid * IDX_PER_W

    @pl.when(sid == 0)
    def _():
        pltpu.sync_copy(tab_hbm, tabS)

    pltpu.sync_copy(tabT_hbm, tabT_v)
    plsc.subcore_barrier()

    iota = jnp.arange(L, dtype=jnp.int32)
    jsp = [jnp.full((L,), j, dtype=jnp.int32) for j in range(EMB_DIM)]

    idxt = (idxt_a, idxt_b)
    idxs = (idxs_a, idxs_b)

    def pair(t, pt):
        base_t = base0 + 2 * t * CHUNK
        base_s = base_t + CHUNK
        it_v = idxt[pt]
        is_v = idxs[pt]

        @pl.when(t == 0)
        def _():
            pltpu.sync_copy(x1_hbm.at[pl.ds(base_s, CHUNK)], is_v)

        @pl.when(t > 0)
        def _():
            pltpu.make_async_copy(
                x1_hbm.at[pl.ds(base_s, CHUNK)], is_v, sem_is).wait()
            pltpu.make_async_copy(
                rows_s, out_hbm.at[pl.ds(base_s, CHUNK)], sem_os).wait()

        @pl.when(t + 1 < NPAIR)
        def _():
            pltpu.async_copy(
                x1_hbm.at[pl.ds(base_s + 2 * CHUNK, CHUNK)],
                idxs[1 - pt], sem_is)

        cps = [
            pltpu.async_copy(
                tabS.at[is_v.at[pl.ds(j * G, G)]],
                rows_s.at[pl.ds(j * G, G)], sem_g)
            for j in range(NG)
        ]

        @pl.when(t == 0)
        def _():
            pltpu.sync_copy(x1_hbm.at[pl.ds(base_t, CHUNK)], it_v)

        @pl.when(t > 0)
        def _():
            pltpu.make_async_copy(
                x1_hbm.at[pl.ds(base_t, CHUNK)], it_v, sem_i).wait()
            pltpu.make_async_copy(
                rows_t, out_hbm.at[pl.ds(base_t, CHUNK)], sem_ot).wait()

        @pl.when(t + 1 < NPAIR)
        def _():
            pltpu.async_copy(
                x1_hbm.at[pl.ds(base_t + 2 * CHUNK, CHUNK)],
                idxt[1 - pt], sem_i)

        def group(g):
            idx_vec = it_v[pl.ds(g * L, L)]
            row_vec = g * L + iota
            for j in range(EMB_DIM):
                vals = plsc.load_gather(tabT_v, [jsp[j], idx_vec])
                plsc.store_scatter(rows_t, [row_vec, jsp[j]], vals)

        @plsc.parallel_loop(0, NVEC, 1, unroll=UNROLL)
        def _(g):
            group(g)

        pltpu.async_copy(rows_t, out_hbm.at[pl.ds(base_t, CHUNK)], sem_ot)

        for cp in cps:
            cp.wait()
        pltpu.async_copy(rows_s, out_hbm.at[pl.ds(base_s, CHUNK)], sem_os)

    def step2(t2, _):
        pair(2 * t2, 0)
        pair(2 * t2 + 1, 1)
        return ()

    lax.fori_loop(0, NPAIR // 2, step2, ())

    last_t = base0 + (STEPS - 2) * CHUNK
    pltpu.make_async_copy(
        rows_t, out_hbm.at[pl.ds(last_t, CHUNK)], sem_ot).wait()
    pltpu.make_async_copy(
        rows_s, out_hbm.at[pl.ds(last_t + CHUNK, CHUNK)], sem_os).wait()


@jax.jit
def _encode(x1d, tab, tabT):
    mesh = plsc.VectorSubcoreMesh(core_axis_name="c", subcore_axis_name="s")
    run = pl.kernel(
        _sc_body,
        out_type=jax.ShapeDtypeStruct((TOTAL, EMB_DIM), jnp.float32),
        mesh=mesh,
        scratch_types=[
            pltpu.VMEM((EMB_DIM, NUM_CHARS), jnp.float32),
            pltpu.VMEM_SHARED((NUM_CHARS, EMB_DIM), jnp.float32),
            pltpu.VMEM((CHUNK,), jnp.int32),
            pltpu.VMEM((CHUNK,), jnp.int32),
            pltpu.VMEM((CHUNK,), jnp.int32),
            pltpu.VMEM((CHUNK,), jnp.int32),
            pltpu.VMEM((CHUNK, EMB_DIM), jnp.float32),
            pltpu.VMEM((CHUNK, EMB_DIM), jnp.float32),
            pltpu.SemaphoreType.DMA,
            pltpu.SemaphoreType.DMA,
            pltpu.SemaphoreType.DMA,
            pltpu.SemaphoreType.DMA,
            pltpu.SemaphoreType.DMA,
        ],
        compiler_params=pltpu.CompilerParams(
            use_tc_tiling_on_sc=False, needs_layout_passes=False),
    )
    return run(x1d, tab, tabT)


def kernel(x, emb_table):
    x1d = jnp.asarray(x, jnp.int32).reshape(TOTAL)
    tabT = emb_table.T.reshape(EMB_DIM, NUM_CHARS)
    y = _encode(x1d, emb_table, tabT)
    return y.reshape(BATCH, MSG_LEN * EMB_DIM)
